# baseline (speedup 1.0000x reference)
import functools

import jax
import jax.numpy as jnp
from jax.experimental import pallas as pl

_B = 256
_D = 256
_S = 1024
_LN_EPS = 1e-5
_SCALE = 1.0 * (_D ** -0.5)
_ROWS = 8
_SUB = 8
_CHUNKS = 4


def _q_kernel(x_ref, g_ref, b_ref, wq_ref, bq_ref, q_ref):
    x = x_ref[...]
    mu = jnp.mean(x, axis=-1, keepdims=True)
    var = jnp.mean((x - mu) ** 2, axis=-1, keepdims=True)
    xn = (x - mu) * jax.lax.rsqrt(var + _LN_EPS) * g_ref[...] + b_ref[...]
    q_ref[...] = (jax.lax.dot_general(
        xn, wq_ref[...], (((1,), (1,)), ((), ())),
        preferred_element_type=jnp.float32) + bq_ref[...]) * _SCALE


def _read_kernel(q_ref, s_ref, m_ref, o_ref, *, rows, sub, chunks):
    csz = sub * _S // chunks
    m = m_ref[...]
    for h in range(rows // sub):
        qb = q_ref[h * sub:(h + 1) * sub]
        base = h * sub * _S
        unnorm = None
        denom = None
        for c in range(chunks):
            sf = s_ref[base + c * csz:base + (c + 1) * csz]
            p = jax.lax.dot_general(
                qb, sf, (((1,), (1,)), ((), ())),
                preferred_element_type=jnp.float32)
            e = jnp.exp(p) * m[:, c * csz:(c + 1) * csz]
            d = jnp.sum(e, axis=1, keepdims=True)
            u = jax.lax.dot_general(
                e, sf, (((1,), (0,)), ((), ())),
                preferred_element_type=jnp.float32)
            unnorm = u if c == 0 else unnorm + u
            denom = d if c == 0 else denom + d
        o_ref[h * sub:(h + 1) * sub] = unnorm * (1.0 / denom)


@jax.jit
def kernel(x, state, ln_gamma, ln_beta, Wq, bq):
    g2 = ln_gamma.reshape(1, _D)
    b2 = ln_beta.reshape(1, _D)
    bq2 = bq.reshape(1, _D)

    q = pl.pallas_call(
        _q_kernel,
        out_shape=jax.ShapeDtypeStruct((_B, _D), jnp.float32),
    )(x, g2, b2, Wq, bq2)

    state2d = state.reshape(_B * _S, _D)
    n = _SUB * _S
    seg = jax.lax.broadcasted_iota(jnp.int32, (_SUB, n), 1) // _S
    row = jax.lax.broadcasted_iota(jnp.int32, (_SUB, n), 0)
    mask = (seg == row).astype(jnp.float32)

    out = pl.pallas_call(
        functools.partial(_read_kernel, rows=_ROWS, sub=_SUB, chunks=_CHUNKS),
        grid=(_B // _ROWS,),
        in_specs=[
            pl.BlockSpec((_ROWS, _D), lambda i: (i, 0)),
            pl.BlockSpec((_ROWS * _S, _D), lambda i: (i, 0)),
            pl.BlockSpec((_SUB, n), lambda i: (0, 0)),
        ],
        out_specs=pl.BlockSpec((_ROWS, _D), lambda i: (i, 0)),
        out_shape=jax.ShapeDtypeStruct((_B, _D), jnp.float32),
    )(q, state2d, mask)
    return out


# speedup vs baseline: 1.1050x; 1.1050x over previous
import functools

import jax
import jax.numpy as jnp
from jax.experimental import pallas as pl
from jax.experimental.pallas import tpu as pltpu

_B = 256
_D = 256
_S = 1024
_LN_EPS = 1e-5
_SCALE = 1.0 * (_D ** -0.5)
_ROWS = 16
_SUB = 8
_CHUNKS = 4


def _read_kernel(x_ref, g_ref, b_ref, wq_ref, bq_ref, s_ref, m_ref,
                 o_ref, q_scr, *, rows, sub, chunks):
    @pl.when(pl.program_id(0) == 0)
    def _():
        x = x_ref[...]
        mu = jnp.mean(x, axis=-1, keepdims=True)
        var = jnp.mean((x - mu) ** 2, axis=-1, keepdims=True)
        xn = (x - mu) * jax.lax.rsqrt(var + _LN_EPS) * g_ref[...] + b_ref[...]
        q_scr[...] = (jax.lax.dot_general(
            xn, wq_ref[...], (((1,), (1,)), ((), ())),
            preferred_element_type=jnp.float32) + bq_ref[...]) * _SCALE

    csz = sub * _S // chunks
    step = pl.program_id(0)
    for h in range(rows // sub):
        qb = q_scr[pl.ds(step * rows + h * sub, sub), :]
        base = h * sub * _S
        unnorm = None
        denom = None
        for c in range(chunks):
            sf = s_ref[base + c * csz:base + (c + 1) * csz]
            p = jax.lax.dot_general(
                qb, sf, (((1,), (1,)), ((), ())),
                preferred_element_type=jnp.float32)
            e = jnp.exp(p) * m_ref[:, c * csz:(c + 1) * csz]
            d = jnp.sum(e, axis=1, keepdims=True)
            u = jax.lax.dot_general(
                e, sf, (((1,), (0,)), ((), ())),
                preferred_element_type=jnp.float32)
            unnorm = u if c == 0 else unnorm + u
            denom = d if c == 0 else denom + d
        o_ref[h * sub:(h + 1) * sub] = unnorm * (1.0 / denom)


@jax.jit
def kernel(x, state, ln_gamma, ln_beta, Wq, bq):
    g2 = ln_gamma.reshape(1, _D)
    b2 = ln_beta.reshape(1, _D)
    bq2 = bq.reshape(1, _D)

    state2d = state.reshape(_B * _S, _D)
    n = _SUB * _S
    seg = jax.lax.broadcasted_iota(jnp.int32, (_SUB, n), 1) // _S
    row = jax.lax.broadcasted_iota(jnp.int32, (_SUB, n), 0)
    mask = (seg == row).astype(jnp.float32)

    const = lambda i: (0, 0)
    out = pl.pallas_call(
        functools.partial(_read_kernel, rows=_ROWS, sub=_SUB, chunks=_CHUNKS),
        grid=(_B // _ROWS,),
        in_specs=[
            pl.BlockSpec((_B, _D), const),
            pl.BlockSpec((1, _D), const),
            pl.BlockSpec((1, _D), const),
            pl.BlockSpec((_D, _D), const),
            pl.BlockSpec((1, _D), const),
            pl.BlockSpec((_ROWS * _S, _D), lambda i: (i, 0)),
            pl.BlockSpec((_SUB, n), const),
        ],
        out_specs=pl.BlockSpec((_ROWS, _D), lambda i: (i, 0)),
        out_shape=jax.ShapeDtypeStruct((_B, _D), jnp.float32),
        scratch_shapes=[pltpu.VMEM((_B, _D), jnp.float32)],
    )(x, g2, b2, Wq, bq2, state2d, mask)
    return out
